# baseline (speedup 1.0000x reference)
import functools

import jax
import jax.numpy as jnp
from jax import lax
from jax.experimental import pallas as pl
from jax.experimental.pallas import tpu as pltpu
from jax.experimental.pallas import tpu_sc as plsc

VOCAB = 1000000
EMBED_DIM = 64
BATCH = 16384
HIST = 50

NUM_CORES = 2
NUM_SUBCORES = 16
NW = NUM_CORES * NUM_SUBCORES
B_TOTAL = BATCH * HIST
B_PER_W = B_TOTAL // NW
CHUNK = 400
NBUF = 4
NCHUNK = B_PER_W // CHUNK


def _gather_kernel(table_hbm, idx_hbm, out_hbm, idx_v, rows_v, *gsems):
    wid = lax.axis_index("s") * NUM_CORES + lax.axis_index("c")
    base = wid * B_PER_W

    pltpu.sync_copy(idx_hbm.at[pl.ds(base, B_PER_W)], idx_v)

    def start(c, buf):
        off = pl.multiple_of(c * CHUNK, CHUNK)
        pltpu.async_copy(table_hbm.at[idx_v.at[pl.ds(off, CHUNK)]],
                         rows_v.at[buf], gsems[buf])

    def wait(buf):
        pltpu.make_async_copy(table_hbm.at[idx_v.at[pl.ds(0, CHUNK)]],
                              rows_v.at[buf], gsems[buf]).wait()

    for b in range(NBUF):
        start(b, b)

    def body(g, carry):
        for b in range(NBUF):
            c = g * NBUF + b
            wait(b)
            pltpu.sync_copy(
                rows_v.at[b],
                out_hbm.at[pl.ds(base + pl.multiple_of(c * CHUNK, CHUNK),
                                 CHUNK)])

            @pl.when(c + NBUF < NCHUNK)
            def _():
                start(c + NBUF, b)

        return carry

    lax.fori_loop(0, NCHUNK // NBUF, body, 0)


@jax.jit
def _lookup(nodes_ids, table):
    idx = nodes_ids.reshape(-1).astype(jnp.int32)
    mesh = plsc.VectorSubcoreMesh(core_axis_name="c", subcore_axis_name="s")
    out = pl.kernel(
        _gather_kernel,
        out_type=jax.ShapeDtypeStruct((B_TOTAL, EMBED_DIM), jnp.float32),
        mesh=mesh,
        scratch_types=[
            pltpu.VMEM((B_PER_W,), jnp.int32),
            pltpu.VMEM((NBUF, CHUNK, EMBED_DIM), jnp.float32),
        ] + [pltpu.SemaphoreType.DMA] * NBUF,
        compiler_params=pltpu.CompilerParams(use_tc_tiling_on_sc=False),
    )(table, idx)
    return out.reshape(BATCH, HIST, EMBED_DIM)


def kernel(nodes_ids, table):
    return _lookup(nodes_ids, table)


# speedup vs baseline: 1.0012x; 1.0012x over previous
import functools

import jax
import jax.numpy as jnp
from jax import lax
from jax.experimental import pallas as pl
from jax.experimental.pallas import tpu as pltpu
from jax.experimental.pallas import tpu_sc as plsc

VOCAB = 1000000
EMBED_DIM = 64
BATCH = 16384
HIST = 50

NUM_CORES = 2
NUM_SUBCORES = 16
NW = NUM_CORES * NUM_SUBCORES
B_PER_W = BATCH // NW
GB = 8
NCHUNK = B_PER_W // GB
NBUF = 2


def _gather_kernel(table_hbm, idx_hbm, out_hbm, idx_v, rows_v,
                   gsem0, gsem1, osem0, osem1):
    gsems = (gsem0, gsem1)
    osems = (osem0, osem1)
    wid = lax.axis_index("s") * NUM_CORES + lax.axis_index("c")
    b0 = wid * B_PER_W

    pltpu.sync_copy(idx_hbm.at[pl.ds(b0, B_PER_W)], idx_v)

    def start_slab(ch, buf):
        for j in range(GB):
            b = ch * GB + j
            pltpu.async_copy(table_hbm.at[idx_v.at[b]],
                             rows_v.at[buf].at[j], gsems[buf])

    def wait_slab(buf):
        for j in range(GB):
            pltpu.make_async_copy(table_hbm.at[idx_v.at[0]],
                                  rows_v.at[buf].at[j], gsems[buf]).wait()

    def start_out(ch, buf):
        pltpu.async_copy(rows_v.at[buf],
                         out_hbm.at[pl.ds(b0 + ch * GB, GB)], osems[buf])

    def wait_out(buf):
        pltpu.make_async_copy(rows_v.at[buf],
                              out_hbm.at[pl.ds(0, GB)], osems[buf]).wait()

    for n in range(NBUF):
        start_slab(n, n)

    def body(g, carry):
        for n in range(NBUF):
            ch = g * NBUF + n
            wait_slab(n)
            start_out(ch, n)

            @pl.when(ch + NBUF < NCHUNK)
            def _():
                wait_out(n)
                start_slab(ch + NBUF, n)

        return carry

    lax.fori_loop(0, NCHUNK // NBUF, body, 0)

    for n in range(NBUF):
        wait_out(n)


@jax.jit
def _lookup(nodes_ids, table):
    mesh = plsc.VectorSubcoreMesh(core_axis_name="c", subcore_axis_name="s")
    return pl.kernel(
        _gather_kernel,
        out_type=jax.ShapeDtypeStruct((BATCH, HIST, EMBED_DIM), jnp.float32),
        mesh=mesh,
        scratch_types=[
            pltpu.VMEM((B_PER_W, HIST), jnp.int32),
            pltpu.VMEM((NBUF, GB, HIST, EMBED_DIM), jnp.float32),
        ] + [pltpu.SemaphoreType.DMA] * (2 * NBUF),
        compiler_params=pltpu.CompilerParams(use_tc_tiling_on_sc=False),
    )(table, nodes_ids.astype(jnp.int32))


def kernel(nodes_ids, table):
    return _lookup(nodes_ids, table)
